# Initial kernel scaffold; baseline (speedup 1.0000x reference)
import jax
import jax.numpy as jnp
from jax.experimental import pallas as pl


def kernel(x, edge_index, W1a, b1a, W1b, b1b, W2a, b2a, W2b, b2b, Wo, bo):
    raise NotImplementedError("write your pallas kernel here")


# speedup vs baseline: 5.2216x; 5.2216x over previous
import functools

import jax
import jax.numpy as jnp
from jax import lax
from jax.experimental import pallas as pl
from jax.experimental.pallas import tpu as pltpu
from jax.experimental.pallas import tpu_sc as plsc

N = 10000
E = 320000
D = 128

NC = 2
NS = 16
NW = NC * NS
EPT = E // NW
CHUNK = 80
NCHUNK = EPT // CHUNK
NTAB = 10240
RPT = NTAB // NS
ZROWS = 128

BN = 2000


@functools.partial(
    pl.kernel,
    out_type=jax.ShapeDtypeStruct((NC * NTAB, D), jnp.float32),
    mesh=plsc.VectorSubcoreMesh(core_axis_name="c", subcore_axis_name="s"),
    scratch_types=[
        pltpu.VMEM((CHUNK,), jnp.int32),
        pltpu.VMEM((CHUNK,), jnp.int32),
        pltpu.VMEM((CHUNK, D), jnp.float32),
        pltpu.VMEM((ZROWS, D), jnp.float32),
        pltpu.VMEM_SHARED((NTAB, D), jnp.float32),
        pltpu.SemaphoreType.DMA,
    ],
)
def _sc_edge_agg(h_hbm, src_hbm, dst_hbm, out_hbm,
                 src_v, dst_v, rows_v, zbuf_v, agg_sh, sem):
    c = lax.axis_index("c")
    s = lax.axis_index("s")
    tid = s * NC + c

    zeros16 = jnp.zeros((16,), jnp.float32)

    def zero_row(i, carry):
        for j in range(D // 16):
            zbuf_v[i, pl.ds(j * 16, 16)] = zeros16
        return carry

    lax.fori_loop(0, ZROWS, zero_row, 0)
    for k in range(RPT // ZROWS):
        pltpu.sync_copy(zbuf_v, agg_sh.at[pl.ds(s * RPT + k * ZROWS, ZROWS)])
    plsc.subcore_barrier()

    base = tid * EPT

    def edge_chunk(i, carry):
        off = pl.multiple_of(base + i * CHUNK, 8)
        pltpu.sync_copy(src_hbm.at[pl.ds(off, CHUNK)], src_v)
        pltpu.sync_copy(dst_hbm.at[pl.ds(off, CHUNK)], dst_v)
        pltpu.async_copy(h_hbm.at[src_v], rows_v, sem).wait()
        pltpu.sync_copy(rows_v, agg_sh.at[dst_v], add=True)
        return carry

    lax.fori_loop(0, NCHUNK, edge_chunk, 0)
    plsc.subcore_barrier()

    row0 = c * NTAB + s * RPT
    pltpu.sync_copy(agg_sh.at[pl.ds(s * RPT, RPT)], out_hbm.at[pl.ds(row0, RPT)])


def _mlp1_body(x_ref, agg_ref, wa_ref, ba_ref, wb_ref, bb_ref, o_ref):
    h = x_ref[...] + agg_ref[0] + agg_ref[1]
    t = jnp.dot(h, wa_ref[...], preferred_element_type=jnp.float32) + ba_ref[...]
    t = jnp.maximum(t, 0.0)
    u = jnp.dot(t, wb_ref[...], preferred_element_type=jnp.float32) + bb_ref[...]
    o_ref[...] = jnp.maximum(u, 0.0)


def _mlp2_body(x_ref, agg_ref, wa_ref, ba_ref, wb_ref, bb_ref,
               wo_ref, bo_ref, o_ref):
    h = x_ref[...] + agg_ref[0] + agg_ref[1]
    t = jnp.dot(h, wa_ref[...], preferred_element_type=jnp.float32) + ba_ref[...]
    t = jnp.maximum(t, 0.0)
    u = jnp.dot(t, wb_ref[...], preferred_element_type=jnp.float32) + bb_ref[...]
    u = jnp.maximum(u, 0.0)
    o_ref[...] = (jnp.dot(u, wo_ref[...], preferred_element_type=jnp.float32)
                  + bo_ref[...])


_row_spec = pl.BlockSpec((BN, D), lambda i: (i, 0))
_agg_spec = pl.BlockSpec((NC, BN, D), lambda i: (0, i, 0))
_w_spec = pl.BlockSpec((D, D), lambda i: (0, 0))
_b_spec = pl.BlockSpec((1, D), lambda i: (0, 0))

_mlp1 = pl.pallas_call(
    _mlp1_body,
    grid=(N // BN,),
    in_specs=[_row_spec, _agg_spec, _w_spec, _b_spec, _w_spec, _b_spec],
    out_specs=_row_spec,
    out_shape=jax.ShapeDtypeStruct((N, D), jnp.float32),
)

_mlp2 = pl.pallas_call(
    _mlp2_body,
    grid=(N // BN,),
    in_specs=[_row_spec, _agg_spec, _w_spec, _b_spec, _w_spec, _b_spec,
              _w_spec, _b_spec],
    out_specs=_row_spec,
    out_shape=jax.ShapeDtypeStruct((N, D), jnp.float32),
)


def kernel(x, edge_index, W1a, b1a, W1b, b1b, W2a, b2a, W2b, b2b, Wo, bo):
    src = edge_index[0]
    dst = edge_index[1]
    agg1 = _sc_edge_agg(x, src, dst).reshape(NC, NTAB, D)
    h1 = _mlp1(x, agg1, W1a, b1a.reshape(1, D), W1b, b1b.reshape(1, D))
    agg2 = _sc_edge_agg(h1, src, dst).reshape(NC, NTAB, D)
    return _mlp2(h1, agg2, W2a, b2a.reshape(1, D), W2b, b2b.reshape(1, D),
                 Wo, bo.reshape(1, D))
